# baseline (speedup 1.0000x reference)
import functools

import jax
import jax.numpy as jnp
from jax import lax
from jax.experimental import pallas as pl
from jax.experimental.pallas import tpu as pltpu
from jax.experimental.pallas import tpu_sc as plsc

TOP_K = 8
N_EXPERTS = 64
IN_FEATURES = 1024
OUT_FEATURES = 64
N_TOKENS = 2048

BLK_N = 128
_PAIRS = N_EXPERTS // 2


def _tc_body(f_ref, a_ref, w_hbm, bflat_ref, allout_ref, gidx_ref,
             wn_vmem, wbf_vmem, sem):
    i = pl.program_id(0)

    @pl.when(i == 0)
    def _load_w():
        cp = pltpu.make_async_copy(w_hbm, wn_vmem, sem)
        cp.start()
        cp.wait()
        wt = jnp.swapaxes(wn_vmem[...], 0, 1)
        wbf_vmem[...] = wt.reshape(
            IN_FEATURES, N_EXPERTS * OUT_FEATURES).astype(jnp.bfloat16)

    acc = jnp.dot(f_ref[...].astype(jnp.bfloat16), wbf_vmem[...],
                  preferred_element_type=jnp.float32)
    acc = acc + bflat_ref[...]
    for p in range(_PAIRS):
        allout_ref[p] = acc[:, p * 128:(p + 1) * 128]

    a = a_ref[...]
    lane = lax.broadcasted_iota(jnp.int32, (BLK_N, N_EXPERTS), 1)
    sel = []
    for _ in range(TOP_K):
        m = jnp.min(a, axis=1, keepdims=True)
        cand = jnp.where(a == m, lane, N_EXPERTS)
        amin = jnp.min(cand, axis=1, keepdims=True)
        sel.append(amin)
        a = jnp.where(lane == amin, jnp.inf, a)

    e = jnp.concatenate(sel, axis=1)
    n = i * BLK_N + lax.broadcasted_iota(jnp.int32, (BLK_N, 1), 0)
    gidx_ref[...] = (e >> 1) * (2 * N_TOKENS) + n * 2 + (e & 1)


_N_PAIRS = N_TOKENS * TOP_K
_CHUNK = 128


def _make_sc_gather():
    info = plsc.get_sparse_core_info()
    nc, ns = info.num_cores, info.num_subcores
    nw = nc * ns
    per_w = _N_PAIRS // nw
    n_chunks = per_w // _CHUNK

    @functools.partial(
        pl.kernel,
        mesh=plsc.VectorSubcoreMesh(core_axis_name="c", subcore_axis_name="s"),
        out_type=jax.ShapeDtypeStruct((_N_PAIRS, OUT_FEATURES), jnp.float32),
        scratch_types=[
            pltpu.VMEM((per_w,), jnp.int32),
            pltpu.VMEM((per_w, OUT_FEATURES), jnp.float32),
            pltpu.SemaphoreType.DMA,
        ],
        compiler_params=pltpu.CompilerParams(use_tc_tiling_on_sc=False),
    )
    def sc_gather(allout_hbm, gidx_hbm, out_hbm, idx_v, rows_v, sem):
        wid = lax.axis_index("s") * nc + lax.axis_index("c")
        base = wid * per_w
        pltpu.sync_copy(gidx_hbm.at[pl.ds(base, per_w)], idx_v)
        copies = []
        for j in range(n_chunks):
            copies.append(pltpu.async_copy(
                allout_hbm.at[idx_v.at[pl.ds(j * _CHUNK, _CHUNK)]],
                rows_v.at[pl.ds(j * _CHUNK, _CHUNK), :],
                sem,
            ))
        for c in copies:
            c.wait()
        pltpu.sync_copy(rows_v, out_hbm.at[pl.ds(base, per_w)])

    return sc_gather


_sc_gather = _make_sc_gather()


@jax.jit
def kernel(features, activated, W, b):
    bflat = b.reshape(1, N_EXPERTS * OUT_FEATURES)

    allout, gidx = pl.pallas_call(
        _tc_body,
        grid=(N_TOKENS // BLK_N,),
        in_specs=[
            pl.BlockSpec((BLK_N, IN_FEATURES), lambda i: (i, 0)),
            pl.BlockSpec((BLK_N, N_EXPERTS), lambda i: (i, 0)),
            pl.BlockSpec(memory_space=pl.ANY),
            pl.BlockSpec((1, N_EXPERTS * OUT_FEATURES), lambda i: (0, 0)),
        ],
        scratch_shapes=[
            pltpu.VMEM((N_EXPERTS, IN_FEATURES, OUT_FEATURES), jnp.float32),
            pltpu.VMEM((IN_FEATURES, N_EXPERTS * OUT_FEATURES), jnp.bfloat16),
            pltpu.SemaphoreType.DMA,
        ],
        out_specs=[
            pl.BlockSpec((_PAIRS, BLK_N, 128), lambda i: (0, i, 0)),
            pl.BlockSpec((BLK_N, TOP_K), lambda i: (i, 0)),
        ],
        out_shape=[
            jax.ShapeDtypeStruct((_PAIRS, N_TOKENS, 128), jnp.float32),
            jax.ShapeDtypeStruct((N_TOKENS, TOP_K), jnp.int32),
        ],
    )(features, activated, W, bflat)

    allout_rows = allout.reshape(2 * N_TOKENS * _PAIRS, OUT_FEATURES)
    out = _sc_gather(allout_rows, gidx.reshape(_N_PAIRS))
    return out.reshape(N_TOKENS, TOP_K, OUT_FEATURES)


# speedup vs baseline: 1.2339x; 1.2339x over previous
import functools

import jax
import jax.numpy as jnp
from jax import lax
from jax.experimental import pallas as pl
from jax.experimental.pallas import tpu as pltpu
from jax.experimental.pallas import tpu_sc as plsc

TOP_K = 8
N_EXPERTS = 64
IN_FEATURES = 1024
OUT_FEATURES = 64
N_TOKENS = 2048

BLK_N = 128
_PAIRS = N_EXPERTS // 2


def _tc_body(f_ref, a_ref, w_ref, bflat_ref, allout_ref, gidx_ref):
    i = pl.program_id(0)
    acc = jnp.dot(f_ref[...].astype(jnp.bfloat16), w_ref[...],
                  preferred_element_type=jnp.float32)
    acc = acc + bflat_ref[...]
    for p in range(_PAIRS):
        allout_ref[p] = acc[:, p * 128:(p + 1) * 128]

    a = a_ref[...]
    lane = lax.broadcasted_iota(jnp.int32, (BLK_N, N_EXPERTS), 1)
    sel = []
    for _ in range(TOP_K):
        m = jnp.min(a, axis=1, keepdims=True)
        cand = jnp.where(a == m, lane, N_EXPERTS)
        amin = jnp.min(cand, axis=1, keepdims=True)
        sel.append(amin)
        a = jnp.where(lane == amin, jnp.inf, a)

    e = jnp.concatenate(sel, axis=1)
    n = i * BLK_N + lax.broadcasted_iota(jnp.int32, (BLK_N, 1), 0)
    gidx_ref[...] = (e >> 1) * (2 * N_TOKENS) + n * 2 + (e & 1)


_N_PAIRS = N_TOKENS * TOP_K
_CHUNK = 128


def _make_sc_gather():
    info = plsc.get_sparse_core_info()
    nc, ns = info.num_cores, info.num_subcores
    nw = nc * ns
    per_w = _N_PAIRS // nw
    n_chunks = per_w // _CHUNK

    @functools.partial(
        pl.kernel,
        mesh=plsc.VectorSubcoreMesh(core_axis_name="c", subcore_axis_name="s"),
        out_type=jax.ShapeDtypeStruct((_N_PAIRS, OUT_FEATURES), jnp.float32),
        scratch_types=[
            pltpu.VMEM((per_w,), jnp.int32),
            pltpu.VMEM((per_w, OUT_FEATURES), jnp.float32),
            pltpu.SemaphoreType.DMA,
        ],
        compiler_params=pltpu.CompilerParams(use_tc_tiling_on_sc=False),
    )
    def sc_gather(allout_hbm, gidx_hbm, out_hbm, idx_v, rows_v, sem):
        wid = lax.axis_index("s") * nc + lax.axis_index("c")
        base = wid * per_w
        pltpu.sync_copy(gidx_hbm.at[pl.ds(base, per_w)], idx_v)
        copies = []
        for j in range(n_chunks):
            copies.append(pltpu.async_copy(
                allout_hbm.at[idx_v.at[pl.ds(j * _CHUNK, _CHUNK)]],
                rows_v.at[pl.ds(j * _CHUNK, _CHUNK), :],
                sem,
            ))
        for c in copies:
            c.wait()
        pltpu.sync_copy(rows_v, out_hbm.at[pl.ds(base, per_w)])

    return sc_gather


_sc_gather = _make_sc_gather()


@jax.jit
def kernel(features, activated, W, b):
    wr = W.transpose(1, 0, 2).reshape(IN_FEATURES, N_EXPERTS * OUT_FEATURES)
    wr = wr.astype(jnp.bfloat16)
    bflat = b.reshape(1, N_EXPERTS * OUT_FEATURES)

    allout, gidx = pl.pallas_call(
        _tc_body,
        grid=(N_TOKENS // BLK_N,),
        in_specs=[
            pl.BlockSpec((BLK_N, IN_FEATURES), lambda i: (i, 0)),
            pl.BlockSpec((BLK_N, N_EXPERTS), lambda i: (i, 0)),
            pl.BlockSpec((IN_FEATURES, N_EXPERTS * OUT_FEATURES),
                         lambda i: (0, 0)),
            pl.BlockSpec((1, N_EXPERTS * OUT_FEATURES), lambda i: (0, 0)),
        ],
        out_specs=[
            pl.BlockSpec((_PAIRS, BLK_N, 128), lambda i: (0, i, 0)),
            pl.BlockSpec((BLK_N, TOP_K), lambda i: (i, 0)),
        ],
        out_shape=[
            jax.ShapeDtypeStruct((_PAIRS, N_TOKENS, 128), jnp.float32),
            jax.ShapeDtypeStruct((N_TOKENS, TOP_K), jnp.int32),
        ],
    )(features, activated, wr, bflat)

    allout_rows = allout.reshape(2 * N_TOKENS * _PAIRS, OUT_FEATURES)
    out = _sc_gather(allout_rows, gidx.reshape(_N_PAIRS))
    return out.reshape(N_TOKENS, TOP_K, OUT_FEATURES)


# speedup vs baseline: 1.3100x; 1.0617x over previous
import functools

import jax
import jax.numpy as jnp
from jax import lax
from jax.experimental import pallas as pl
from jax.experimental.pallas import tpu as pltpu

TOP_K = 8
N_EXPERTS = 64
IN_FEATURES = 1024
OUT_FEATURES = 64
N_TOKENS = 2048

BLK_N = 128
_NBLK = N_TOKENS // BLK_N


def _body(f_ref, a_ref, w_ref, bflat_ref, out_ref, acc_buf):
    s = pl.program_id(0)

    @pl.when(s < _NBLK)
    def _produce():
        f = f_ref[...].astype(jnp.bfloat16)
        acc_buf[s % 2] = jnp.dot(f, w_ref[...],
                                 preferred_element_type=jnp.float32)

    @pl.when(s > 0)
    def _consume():
        acc = acc_buf[(s - 1) % 2] + bflat_ref[...]

        a = a_ref[...]
        lane = lax.broadcasted_iota(jnp.int32, (BLK_N, N_EXPERTS), 1)
        sel = []
        for _ in range(TOP_K):
            m = jnp.min(a, axis=1, keepdims=True)
            cand = jnp.where(a == m, lane, N_EXPERTS)
            amin = jnp.min(cand, axis=1, keepdims=True)
            sel.append(amin)
            a = jnp.where(lane == amin, jnp.inf, a)

        for k in range(TOP_K):
            e = sel[k]
            cur = acc
            width = (N_EXPERTS // 2) * OUT_FEATURES
            for bit in range(5, -1, -1):
                take_hi = ((e >> bit) & 1) == 1
                cur = jnp.where(take_hi, cur[:, width:], cur[:, :width])
                width //= 2
            out_ref[:, k * OUT_FEATURES:(k + 1) * OUT_FEATURES] = cur


@jax.jit
def kernel(features, activated, W, b):
    wr = W.transpose(1, 0, 2).reshape(IN_FEATURES, N_EXPERTS * OUT_FEATURES)
    wr = wr.astype(jnp.bfloat16)
    bflat = b.reshape(1, N_EXPERTS * OUT_FEATURES)

    out = pl.pallas_call(
        _body,
        grid=(_NBLK + 1,),
        in_specs=[
            pl.BlockSpec((BLK_N, IN_FEATURES),
                         lambda s: (jnp.minimum(s, _NBLK - 1), 0)),
            pl.BlockSpec((BLK_N, N_EXPERTS),
                         lambda s: (jnp.maximum(s - 1, 0), 0)),
            pl.BlockSpec((IN_FEATURES, N_EXPERTS * OUT_FEATURES),
                         lambda s: (0, 0)),
            pl.BlockSpec((1, N_EXPERTS * OUT_FEATURES), lambda s: (0, 0)),
        ],
        out_specs=pl.BlockSpec((BLK_N, TOP_K * OUT_FEATURES),
                               lambda s: (jnp.maximum(s - 1, 0), 0)),
        out_shape=jax.ShapeDtypeStruct(
            (N_TOKENS, TOP_K * OUT_FEATURES), jnp.float32),
        scratch_shapes=[
            pltpu.VMEM((2, BLK_N, N_EXPERTS * OUT_FEATURES), jnp.float32),
        ],
    )(features, activated, wr, bflat)
    return out.reshape(N_TOKENS, TOP_K, OUT_FEATURES)


# speedup vs baseline: 1.5330x; 1.1702x over previous
import functools

import jax
import jax.numpy as jnp
from jax import lax
from jax.experimental import pallas as pl
from jax.experimental.pallas import tpu as pltpu

TOP_K = 8
N_EXPERTS = 64
IN_FEATURES = 1024
OUT_FEATURES = 64
N_TOKENS = 2048

BLK_N = 128
_NBLK = N_TOKENS // BLK_N


def _body(f_ref, a_ref, w_ref, bflat_ref, out_ref, acc_a, acc_b):
    s = pl.program_id(0)

    def produce(buf):
        f = f_ref[...].astype(jnp.bfloat16)
        buf[...] = jnp.dot(f, w_ref[...], preferred_element_type=jnp.float32)

    def consume(buf):
        acc = buf[...] + bflat_ref[...]

        a = a_ref[...]
        lane = lax.broadcasted_iota(jnp.int32, (BLK_N, N_EXPERTS), 1)
        sel = []
        for _ in range(TOP_K):
            m = jnp.min(a, axis=1, keepdims=True)
            cand = jnp.where(a == m, lane, N_EXPERTS)
            amin = jnp.min(cand, axis=1, keepdims=True)
            sel.append(amin)
            a = jnp.where(lane == amin, jnp.inf, a)

        for k in range(TOP_K):
            e = sel[k]
            cur = acc
            width = (N_EXPERTS // 2) * OUT_FEATURES
            for bit in range(5, -1, -1):
                take_hi = ((e >> bit) & 1) == 1
                cur = jnp.where(take_hi, cur[:, width:], cur[:, :width])
                width //= 2
            out_ref[:, k * OUT_FEATURES:(k + 1) * OUT_FEATURES] = cur

    @pl.when(s % 2 == 0)
    def _even():
        produce(acc_a)
        consume(acc_b)

    @pl.when(s % 2 == 1)
    def _odd():
        produce(acc_b)
        consume(acc_a)


@jax.jit
def kernel(features, activated, W, b):
    wr = W.transpose(1, 0, 2).reshape(IN_FEATURES, N_EXPERTS * OUT_FEATURES)
    wr = wr.astype(jnp.bfloat16)
    bflat = b.reshape(1, N_EXPERTS * OUT_FEATURES)

    out = pl.pallas_call(
        _body,
        grid=(_NBLK + 1,),
        in_specs=[
            pl.BlockSpec((BLK_N, IN_FEATURES),
                         lambda s: (jnp.minimum(s, _NBLK - 1), 0)),
            pl.BlockSpec((BLK_N, N_EXPERTS),
                         lambda s: (jnp.maximum(s - 1, 0), 0)),
            pl.BlockSpec((IN_FEATURES, N_EXPERTS * OUT_FEATURES),
                         lambda s: (0, 0)),
            pl.BlockSpec((1, N_EXPERTS * OUT_FEATURES), lambda s: (0, 0)),
        ],
        out_specs=pl.BlockSpec((BLK_N, TOP_K * OUT_FEATURES),
                               lambda s: (jnp.maximum(s - 1, 0), 0)),
        out_shape=jax.ShapeDtypeStruct(
            (N_TOKENS, TOP_K * OUT_FEATURES), jnp.float32),
        scratch_shapes=[
            pltpu.VMEM((BLK_N, N_EXPERTS * OUT_FEATURES), jnp.float32),
            pltpu.VMEM((BLK_N, N_EXPERTS * OUT_FEATURES), jnp.float32),
        ],
    )(features, activated, wr, bflat)
    return out.reshape(N_TOKENS, TOP_K, OUT_FEATURES)


# speedup vs baseline: 1.6302x; 1.0634x over previous
import functools

import jax
import jax.numpy as jnp
from jax import lax
from jax.experimental import pallas as pl
from jax.experimental.pallas import tpu as pltpu

TOP_K = 8
N_EXPERTS = 64
IN_FEATURES = 1024
OUT_FEATURES = 64
N_TOKENS = 2048

BLK_N = 128
_NBLK = N_TOKENS // BLK_N


def _body(f_ref, a_ref, w_ref, bflat_ref, out_ref, acc_a, acc_b):
    s = pl.program_id(0)

    def produce(buf):
        f = f_ref[...].astype(jnp.bfloat16)
        buf[...] = jnp.dot(f, w_ref[...],
                           preferred_element_type=jnp.float32)

    def consume(buf):
        acc = buf[...].astype(jnp.bfloat16) + bflat_ref[...]

        a = a_ref[...]
        lane = lax.broadcasted_iota(jnp.int32, (BLK_N, N_EXPERTS), 1)
        sel = []
        for _ in range(TOP_K):
            m = jnp.min(a, axis=1, keepdims=True)
            cand = jnp.where(a == m, lane, N_EXPERTS)
            amin = jnp.min(cand, axis=1, keepdims=True)
            sel.append(amin)
            a = jnp.where(lane == amin, jnp.inf, a)

        for k in range(TOP_K):
            e = sel[k]
            cur = acc
            width = (N_EXPERTS // 2) * OUT_FEATURES
            for bit in range(5, -1, -1):
                take_hi = ((e >> bit) & 1) == 1
                cur = jnp.where(take_hi, cur[:, width:], cur[:, :width])
                width //= 2
            out_ref[:, k * OUT_FEATURES:(k + 1) * OUT_FEATURES] = (
                cur.astype(jnp.float32))

    @pl.when(s % 2 == 0)
    def _even():
        produce(acc_a)
        consume(acc_b)

    @pl.when(s % 2 == 1)
    def _odd():
        produce(acc_b)
        consume(acc_a)


@jax.jit
def kernel(features, activated, W, b):
    wr = W.transpose(1, 0, 2).reshape(IN_FEATURES, N_EXPERTS * OUT_FEATURES)
    wr = wr.astype(jnp.bfloat16)
    bflat = b.reshape(1, N_EXPERTS * OUT_FEATURES).astype(jnp.bfloat16)

    out = pl.pallas_call(
        _body,
        grid=(_NBLK + 1,),
        in_specs=[
            pl.BlockSpec((BLK_N, IN_FEATURES),
                         lambda s: (jnp.minimum(s, _NBLK - 1), 0)),
            pl.BlockSpec((BLK_N, N_EXPERTS),
                         lambda s: (jnp.maximum(s - 1, 0), 0)),
            pl.BlockSpec((IN_FEATURES, N_EXPERTS * OUT_FEATURES),
                         lambda s: (0, 0)),
            pl.BlockSpec((1, N_EXPERTS * OUT_FEATURES), lambda s: (0, 0)),
        ],
        out_specs=pl.BlockSpec((BLK_N, TOP_K * OUT_FEATURES),
                               lambda s: (jnp.maximum(s - 1, 0), 0)),
        out_shape=jax.ShapeDtypeStruct(
            (N_TOKENS, TOP_K * OUT_FEATURES), jnp.float32),
        scratch_shapes=[
            pltpu.VMEM((BLK_N, N_EXPERTS * OUT_FEATURES), jnp.float32),
            pltpu.VMEM((BLK_N, N_EXPERTS * OUT_FEATURES), jnp.float32),
        ],
    )(features, activated, wr, bflat)
    return out.reshape(N_TOKENS, TOP_K, OUT_FEATURES)


# speedup vs baseline: 1.6944x; 1.0394x over previous
import functools

import jax
import jax.numpy as jnp
from jax import lax
from jax.experimental import pallas as pl
from jax.experimental.pallas import tpu as pltpu

TOP_K = 8
N_EXPERTS = 64
IN_FEATURES = 1024
OUT_FEATURES = 64
N_TOKENS = 2048

BLK_N = 256
_NBLK = N_TOKENS // BLK_N


def _body(f_ref, a_ref, w_ref, bflat_ref, out_ref, acc_a, acc_b):
    s = pl.program_id(0)

    def produce(buf):
        f = f_ref[...].astype(jnp.bfloat16)
        buf[...] = jnp.dot(f, w_ref[...],
                           preferred_element_type=jnp.float32)

    def consume(buf):
        acc = buf[...].astype(jnp.bfloat16) + bflat_ref[...]

        a = a_ref[...]
        lane = lax.broadcasted_iota(jnp.int32, (BLK_N, N_EXPERTS), 1)
        sel = []
        for _ in range(TOP_K):
            m = jnp.min(a, axis=1, keepdims=True)
            cand = jnp.where(a == m, lane, N_EXPERTS)
            amin = jnp.min(cand, axis=1, keepdims=True)
            sel.append(amin)
            a = jnp.where(lane == amin, jnp.inf, a)

        for k in range(TOP_K):
            e = sel[k]
            cur = acc
            width = (N_EXPERTS // 2) * OUT_FEATURES
            for bit in range(5, -1, -1):
                take_hi = ((e >> bit) & 1) == 1
                cur = jnp.where(take_hi, cur[:, width:], cur[:, :width])
                width //= 2
            out_ref[:, k * OUT_FEATURES:(k + 1) * OUT_FEATURES] = (
                cur.astype(jnp.float32))

    @pl.when(s % 2 == 0)
    def _even():
        produce(acc_a)
        consume(acc_b)

    @pl.when(s % 2 == 1)
    def _odd():
        produce(acc_b)
        consume(acc_a)


@jax.jit
def kernel(features, activated, W, b):
    wr = W.transpose(1, 0, 2).reshape(IN_FEATURES, N_EXPERTS * OUT_FEATURES)
    wr = wr.astype(jnp.bfloat16)
    bflat = b.reshape(1, N_EXPERTS * OUT_FEATURES).astype(jnp.bfloat16)

    out = pl.pallas_call(
        _body,
        grid=(_NBLK + 1,),
        in_specs=[
            pl.BlockSpec((BLK_N, IN_FEATURES),
                         lambda s: (jnp.minimum(s, _NBLK - 1), 0)),
            pl.BlockSpec((BLK_N, N_EXPERTS),
                         lambda s: (jnp.maximum(s - 1, 0), 0)),
            pl.BlockSpec((IN_FEATURES, N_EXPERTS * OUT_FEATURES),
                         lambda s: (0, 0)),
            pl.BlockSpec((1, N_EXPERTS * OUT_FEATURES), lambda s: (0, 0)),
        ],
        out_specs=pl.BlockSpec((BLK_N, TOP_K * OUT_FEATURES),
                               lambda s: (jnp.maximum(s - 1, 0), 0)),
        out_shape=jax.ShapeDtypeStruct(
            (N_TOKENS, TOP_K * OUT_FEATURES), jnp.float32),
        scratch_shapes=[
            pltpu.VMEM((BLK_N, N_EXPERTS * OUT_FEATURES), jnp.float32),
            pltpu.VMEM((BLK_N, N_EXPERTS * OUT_FEATURES), jnp.float32),
        ],
    )(features, activated, wr, bflat)
    return out.reshape(N_TOKENS, TOP_K, OUT_FEATURES)
